# Initial kernel scaffold; baseline (speedup 1.0000x reference)
import functools

import jax
import jax.numpy as jnp
from jax import lax
from jax.experimental import pallas as pl
from jax.experimental.pallas import tpu as pltpu
from jax.experimental.pallas import tpu_sc as plsc

N = 10000
E = 320000
D = 128

NC = 2
NS = 16
NW = NC * NS
EPW = E // NW
CHUNK = 80
NCHUNK = EPW // CHUNK
RPW = N // NS
WB = 125
NWB = RPW // WB


def _matmul_body(x_ref, w_ref, o_ref):
    o_ref[...] = jnp.dot(x_ref[...], w_ref[...],
                         preferred_element_type=jnp.float32,
                         precision=jax.lax.Precision.HIGHEST)


def _matmul(x, w):
    return pl.pallas_call(
        _matmul_body,
        grid=(10,),
        in_specs=[
            pl.BlockSpec((N // 10, D), lambda i: (i, 0)),
            pl.BlockSpec((D, D), lambda i: (0, 0)),
        ],
        out_specs=pl.BlockSpec((N // 10, D), lambda i: (i, 0)),
        out_shape=jax.ShapeDtypeStruct((N, D), jnp.float32),
    )(x, w)


def _combine_body(p_ref, b_ref, o_ref):
    s = p_ref[0] + p_ref[1] + b_ref[...]
    o_ref[...] = jnp.maximum(s, 0.0)


def _combine(partials, b2):
    return pl.pallas_call(
        _combine_body,
        grid=(10,),
        in_specs=[
            pl.BlockSpec((NC, N // 10, D), lambda i: (0, i, 0)),
            pl.BlockSpec((1, D), lambda i: (0, 0)),
        ],
        out_specs=pl.BlockSpec((N // 10, D), lambda i: (i, 0)),
        out_shape=jax.ShapeDtypeStruct((N, D), jnp.float32),
    )(partials, b2)


def _sc_body(xw_hbm, row_hbm, col_hbm, ew_hbm, out_hbm,
             acc, ridx_v, cidx_v, ew_v, rows_v, zbuf_v):
    c = lax.axis_index("c")
    s = lax.axis_index("s")
    wid = c * NS + s
    zero = jnp.zeros((16,), jnp.float32)

    @pl.loop(0, WB)
    def _(i):
        @pl.loop(0, D, step=16)
        def _(j):
            zbuf_v[i, pl.ds(j, 16)] = zero

    @pl.loop(0, NWB)
    def _(k):
        pltpu.sync_copy(zbuf_v, acc.at[pl.ds(s * RPW + k * WB, WB)])

    plsc.subcore_barrier()

    pltpu.sync_copy(row_hbm.at[wid], ridx_v)
    pltpu.sync_copy(col_hbm.at[wid], cidx_v)
    pltpu.sync_copy(ew_hbm.at[wid], ew_v)

    @pl.loop(0, NCHUNK)
    def _(g):
        pltpu.sync_copy(xw_hbm.at[cidx_v.at[g]], rows_v)

        @pl.loop(0, CHUNK // 16)
        def _(q):
            wv = ew_v[g, pl.ds(q * 16, 16)]
            for e in range(16):
                we = jnp.take(wv, jnp.full((16,), e, jnp.int32),
                              mode="promise_in_bounds")
                er = q * 16 + e
                for j in range(D // 16):
                    rows_v[er, pl.ds(j * 16, 16)] = (
                        rows_v[er, pl.ds(j * 16, 16)] * we)

        pltpu.sync_copy(rows_v, acc.at[ridx_v.at[g]], add=True)

    plsc.subcore_barrier()

    @pl.loop(0, NWB)
    def _(k):
        base = s * RPW + k * WB
        pltpu.sync_copy(acc.at[pl.ds(base, WB)], zbuf_v)
        pltpu.sync_copy(zbuf_v, out_hbm.at[c].at[pl.ds(base, WB)])


@functools.partial(
    pl.kernel,
    out_type=jax.ShapeDtypeStruct((NC, N, D), jnp.float32),
    mesh=plsc.VectorSubcoreMesh(core_axis_name="c", subcore_axis_name="s"),
    scratch_types=[
        pltpu.VMEM_SHARED((N, D), jnp.float32),
        pltpu.VMEM((NCHUNK, CHUNK), jnp.int32),
        pltpu.VMEM((NCHUNK, CHUNK), jnp.int32),
        pltpu.VMEM((NCHUNK, CHUNK), jnp.float32),
        pltpu.VMEM((CHUNK, D), jnp.float32),
        pltpu.VMEM((WB, D), jnp.float32),
    ],
)
def _sc_aggregate(xw_hbm, row_hbm, col_hbm, ew_hbm, out_hbm,
                  acc, ridx_v, cidx_v, ew_v, rows_v, zbuf_v):
    _sc_body(xw_hbm, row_hbm, col_hbm, ew_hbm, out_hbm,
             acc, ridx_v, cidx_v, ew_v, rows_v, zbuf_v)


def kernel(x, edge_index, edge_weight, W, b):
    xw = _matmul(x, W)
    row3 = edge_index[0].reshape(NW, NCHUNK, CHUNK)
    col3 = edge_index[1].reshape(NW, NCHUNK, CHUNK)
    ew3 = edge_weight.reshape(NW, NCHUNK, CHUNK)
    partials = _sc_aggregate(xw, row3, col3, ew3)
    return _combine(partials, b.reshape(1, D))


# speedup vs baseline: 6.4096x; 6.4096x over previous
import functools

import jax
import jax.numpy as jnp
from jax import lax
from jax.experimental import pallas as pl
from jax.experimental.pallas import tpu as pltpu
from jax.experimental.pallas import tpu_sc as plsc

N = 10000
E = 320000
D = 128

NC = 2
NS = 16
NW = NC * NS
EPW = E // NW
CHUNK = 80
NCHUNK = EPW // CHUNK
SBCH = 25
NSB = NCHUNK // SBCH
NPAD = 10240
RPW = NPAD // NS
WB = 80
NWB = RPW // WB


def _matmul_body(x_ref, w_ref, o_ref):
    o_ref[...] = jnp.dot(x_ref[...], w_ref[...],
                         preferred_element_type=jnp.float32,
                         precision=jax.lax.Precision.HIGHEST)


def _matmul(x, w):
    return pl.pallas_call(
        _matmul_body,
        grid=(10,),
        in_specs=[
            pl.BlockSpec((N // 10, D), lambda i: (i, 0)),
            pl.BlockSpec((D, D), lambda i: (0, 0)),
        ],
        out_specs=pl.BlockSpec((N // 10, D), lambda i: (i, 0)),
        out_shape=jax.ShapeDtypeStruct((N, D), jnp.float32),
    )(x, w)


def _combine_body(p_ref, b_ref, o_ref):
    s = p_ref[0] + p_ref[1] + b_ref[...]
    o_ref[...] = jnp.maximum(s, 0.0)


def _combine(partials, b2):
    return pl.pallas_call(
        _combine_body,
        grid=(10,),
        in_specs=[
            pl.BlockSpec((NC, N // 10, D), lambda i: (0, i, 0)),
            pl.BlockSpec((1, D), lambda i: (0, 0)),
        ],
        out_specs=pl.BlockSpec((N // 10, D), lambda i: (i, 0)),
        out_shape=jax.ShapeDtypeStruct((N, D), jnp.float32),
    )(partials, b2)


def _bcast_lane(vec, lane):
    idx = jnp.full((16, 1), lane, jnp.int32)
    return lax.gather(
        vec, idx,
        dimension_numbers=lax.GatherDimensionNumbers(
            offset_dims=(), collapsed_slice_dims=(0,), start_index_map=(0,)),
        slice_sizes=(1,),
        mode=lax.GatherScatterMode.PROMISE_IN_BOUNDS)


def _sc_body(xw_hbm, row_hbm, col_hbm, ew_hbm, out_hbm,
             acc, ridx_v, cidx_v, ew_v, rows_v, zbuf_v):
    c = lax.axis_index("c")
    s = lax.axis_index("s")
    wid = c * NS + s
    zero = jnp.zeros((16,), jnp.float32)

    @pl.loop(0, WB)
    def _(i):
        @pl.loop(0, D, step=16)
        def _(j):
            zbuf_v[i, pl.ds(j, 16)] = zero

    @pl.loop(0, NWB)
    def _(k):
        pltpu.sync_copy(zbuf_v, acc.at[pl.ds(s * RPW + k * WB, WB)])

    plsc.subcore_barrier()

    @pl.loop(0, NSB)
    def _(sb):
        pltpu.sync_copy(row_hbm.at[wid, sb], ridx_v)
        pltpu.sync_copy(col_hbm.at[wid, sb], cidx_v)
        pltpu.sync_copy(ew_hbm.at[wid, sb], ew_v)

        @pl.loop(0, SBCH)
        def _(g):
            pltpu.sync_copy(xw_hbm.at[cidx_v.at[g]], rows_v)

            @pl.loop(0, CHUNK // 16)
            def _(q):
                wv = ew_v[g, pl.ds(q * 16, 16)]
                for e in range(16):
                    we = _bcast_lane(wv, e)
                    er = q * 16 + e
                    for j in range(D // 16):
                        rows_v[er, pl.ds(j * 16, 16)] = (
                            rows_v[er, pl.ds(j * 16, 16)] * we)

            pltpu.sync_copy(rows_v, acc.at[ridx_v.at[g]], add=True)

    plsc.subcore_barrier()

    @pl.loop(0, NWB)
    def _(k):
        base = s * RPW + k * WB
        pltpu.sync_copy(acc.at[pl.ds(base, WB)], zbuf_v)
        pltpu.sync_copy(zbuf_v, out_hbm.at[c].at[pl.ds(base, WB)])


@functools.partial(
    pl.kernel,
    out_type=jax.ShapeDtypeStruct((NC, NPAD, D), jnp.float32),
    mesh=plsc.VectorSubcoreMesh(core_axis_name="c", subcore_axis_name="s"),
    scratch_types=[
        pltpu.VMEM_SHARED((NPAD, D), jnp.float32),
        pltpu.VMEM((SBCH, CHUNK), jnp.int32),
        pltpu.VMEM((SBCH, CHUNK), jnp.int32),
        pltpu.VMEM((SBCH, CHUNK), jnp.float32),
        pltpu.VMEM((CHUNK, D), jnp.float32),
        pltpu.VMEM((WB, D), jnp.float32),
    ],
)
def _sc_aggregate(xw_hbm, row_hbm, col_hbm, ew_hbm, out_hbm,
                  acc, ridx_v, cidx_v, ew_v, rows_v, zbuf_v):
    _sc_body(xw_hbm, row_hbm, col_hbm, ew_hbm, out_hbm,
             acc, ridx_v, cidx_v, ew_v, rows_v, zbuf_v)


def kernel(x, edge_index, edge_weight, W, b):
    xw = _matmul(x, W)
    row3 = edge_index[0].reshape(NW, NSB, SBCH, CHUNK)
    col3 = edge_index[1].reshape(NW, NSB, SBCH, CHUNK)
    ew3 = edge_weight.reshape(NW, NSB, SBCH, CHUNK)
    partials = _sc_aggregate(xw, row3, col3, ew3)
    return _combine(partials, b.reshape(1, D))
